# baseline (speedup 1.0000x reference)
import dataclasses
from functools import partial

import jax
import jax.numpy as jnp
from jax.experimental import pallas as pl
from jax.experimental.pallas import tpu as pltpu
from jax.experimental.pallas import tpu_sc as plsc

B = 4096
F = 2048
K = 1024
D = 64
NJ = F // D
N_CODES = B * NJ
BM = 512
GRID_M = B // BM
SC_CORES = 2
SC_SUBCORES = 16
SC_WORKERS = SC_CORES * SC_SUBCORES
GATHER_W = 256

_DOT_STD = (((1,), (0,)), ((), ()))
_DOT_TB = (((1,), (1,)), ((), ()))


def _encode_body(x_ref, w1_ref, b1_ref, w2_ref, b2_ref, lat_ref):
    xb = x_ref[...].astype(jnp.bfloat16)
    h = jax.lax.dot_general(xb, w1_ref[...], _DOT_TB,
                            preferred_element_type=jnp.float32) + b1_ref[...]
    hb = jnp.maximum(h, 0.0).astype(jnp.bfloat16)
    lat_ref[...] = jax.lax.dot_general(
        hb, w2_ref[...], _DOT_TB,
        preferred_element_type=jnp.float32) + b2_ref[...]


def _encode(x, w1b, b1, w2b, b2):
    return pl.pallas_call(
        _encode_body,
        grid=(GRID_M,),
        in_specs=[
            pl.BlockSpec((BM, F), lambda m: (m, 0)),
            pl.BlockSpec((F, F), lambda m: (0, 0)),
            pl.BlockSpec((1, F), lambda m: (0, 0)),
            pl.BlockSpec((F, F), lambda m: (0, 0)),
            pl.BlockSpec((1, F), lambda m: (0, 0)),
        ],
        out_specs=pl.BlockSpec((BM, F), lambda m: (m, 0)),
        out_shape=jax.ShapeDtypeStruct((B, F), jnp.float32),
        compiler_params=pltpu.CompilerParams(
            dimension_semantics=("arbitrary",)),
    )(x, w1b, b1, w2b, b2)


BC = 4096
VQ_STEPS = N_CODES // BC


VQ_RT = 256


def _vq_body(codes_ref, cbt2_ref, cn_ref, idx_ref):
    cbtb = cbt2_ref[...].astype(jnp.bfloat16)
    cn = cn_ref[...]
    for rt in range(BC // VQ_RT):
        cs = codes_ref[pl.ds(rt * VQ_RT, VQ_RT), :].astype(jnp.bfloat16)
        d = jax.lax.dot_general(cs, cbtb, (((1,), (0,)), ((), ())),
                                preferred_element_type=jnp.float32)
        ij = jnp.argmin(d + cn, axis=1).astype(jnp.int32)
        idx_ref[0, 0, pl.ds(rt * VQ_RT, VQ_RT)] = ij


def _vq(codes, cbt2, cn):
    steps = codes.shape[0] // BC
    return pl.pallas_call(
        _vq_body,
        grid=(steps,),
        in_specs=[
            pl.BlockSpec((BC, D), lambda i: (i, 0)),
            pl.BlockSpec((D, K), lambda i: (0, 0)),
            pl.BlockSpec((1, K), lambda i: (0, 0)),
        ],
        out_specs=pl.BlockSpec((1, 1, BC), lambda i: (i, 0, 0)),
        out_shape=jax.ShapeDtypeStruct((steps, 1, BC), jnp.int32),
        compiler_params=pltpu.CompilerParams(
            dimension_semantics=("arbitrary",)),
    )(codes, cbt2, cn)


def _sc_gather_hist(idx2d, cb_pad):
    cp = pltpu.CompilerParams()
    if "needs_layout_passes" in pltpu.CompilerParams.__dataclass_fields__:
        cp = dataclasses.replace(cp, needs_layout_passes=False)
    mesh = plsc.VectorSubcoreMesh(core_axis_name="c", subcore_axis_name="s")
    n_idx = idx2d.shape[1]
    out_types = (
        jax.ShapeDtypeStruct((n_idx, 128), jnp.float32),
        jax.ShapeDtypeStruct((SC_WORKERS, K), jnp.float32),
    )

    @partial(pl.kernel, out_type=out_types, mesh=mesh,
             scratch_types=[pltpu.VMEM((K,), jnp.float32)],
             compiler_params=cp)
    def k(cb_hbm, i_hbm, q_hbm, h_hbm, hist_ref):
        @pl.loop(0, K, step=16)
        def _(c):
            hist_ref[pl.ds(c, 16)] = jnp.zeros((16,), jnp.float32)

        def body(i_vmem, o_vmem):
            pltpu.sync_copy(cb_hbm.at[i_vmem.at[0]], o_vmem)

            @pl.loop(0, GATHER_W, step=16)
            def _(c):
                v = i_vmem[0, pl.ds(c, 16)]
                plsc.addupdate_scatter(hist_ref, [v],
                                       jnp.ones((16,), jnp.float32))

        pltpu.emit_pipeline(
            body,
            grid=(n_idx // GATHER_W,),
            in_specs=[pl.BlockSpec((1, GATHER_W), index_map=lambda i: (0, i))],
            out_specs=[pl.BlockSpec((GATHER_W, 128),
                                    index_map=lambda i: (i, 0))],
            core_axis_name=("c", "s"),
            dimension_semantics=(pltpu.PARALLEL,),
        )(i_hbm, q_hbm)

        sid = jax.lax.axis_index("c") * SC_SUBCORES + jax.lax.axis_index("s")
        pltpu.sync_copy(hist_ref, h_hbm.at[sid])

    return k(cb_pad, idx2d)


def _decode_body(q_ref, lat_ref, w3_ref, b3_ref, w4_ref, b4_ref, part_ref,
                 rec_ref, vq_ref, ppl_ref, acc_ref):
    m = pl.program_id(0)

    @pl.when(m == 0)
    def _():
        acc_ref[0] = 0.0

    qb = q_ref[...]
    diff = qb - lat_ref[...]
    acc_ref[0] += jnp.sum(diff * diff)

    h2 = jax.lax.dot_general(qb.astype(jnp.bfloat16), w3_ref[...], _DOT_TB,
                             preferred_element_type=jnp.float32) + b3_ref[...]
    h2b = jnp.maximum(h2, 0.0).astype(jnp.bfloat16)
    rec_ref[...] = jax.lax.dot_general(h2b, w4_ref[...], _DOT_TB,
                                       preferred_element_type=jnp.float32
                                       ) + b4_ref[...]

    @pl.when(m == GRID_M - 1)
    def _():
        vq_ref[...] = jnp.reshape(acc_ref[0] * (1.25 / float(N_CODES * D)),
                                  (1, 1))
        counts = jnp.sum(part_ref[...], axis=0)
        p = counts * (1.0 / float(N_CODES))
        ppl_ref[...] = jnp.reshape(jnp.exp(-jnp.sum(p * jnp.log(p + 1e-10))),
                                   (1, 1))


def _decode(q, latent, w3b, b3, w4b, b4, partials):
    return pl.pallas_call(
        _decode_body,
        grid=(GRID_M,),
        in_specs=[
            pl.BlockSpec((BM, F), lambda m: (m, 0)),
            pl.BlockSpec((BM, F), lambda m: (m, 0)),
            pl.BlockSpec((F, F), lambda m: (0, 0)),
            pl.BlockSpec((1, F), lambda m: (0, 0)),
            pl.BlockSpec((F, F), lambda m: (0, 0)),
            pl.BlockSpec((1, F), lambda m: (0, 0)),
            pl.BlockSpec((2 * SC_WORKERS, K), lambda m: (0, 0)),
        ],
        out_specs=[
            pl.BlockSpec((BM, F), lambda m: (m, 0)),
            pl.BlockSpec((1, 1), lambda m: (0, 0)),
            pl.BlockSpec((1, 1), lambda m: (0, 0)),
        ],
        out_shape=[
            jax.ShapeDtypeStruct((B, F), jnp.float32),
            jax.ShapeDtypeStruct((1, 1), jnp.float32),
            jax.ShapeDtypeStruct((1, 1), jnp.float32),
        ],
        scratch_shapes=[pltpu.SMEM((1,), jnp.float32)],
        compiler_params=pltpu.CompilerParams(
            dimension_semantics=("arbitrary",)),
    )(q, latent, w3b, b3, w4b, b4, partials)


def kernel(x, W1, b1, W2, b2, W3, b3, W4, b4, codebook):
    w1b = W1.astype(jnp.bfloat16)
    w2b = W2.astype(jnp.bfloat16)
    w3b = W3.astype(jnp.bfloat16)
    w4b = W4.astype(jnp.bfloat16)
    b1r = b1.reshape(1, F)
    b2r = b2.reshape(1, F)
    b3r = b3.reshape(1, F)
    b4r = b4.reshape(1, F)

    cbt2 = codebook.T * jnp.float32(-2.0)
    cn = jnp.sum(codebook * codebook, axis=1)[None, :]
    cb_pad = jnp.pad(codebook, ((0, 0), (0, 128 - D)))

    latent = _encode(x, w1b, b1r, w2b, b2r)
    codes = latent.reshape(N_CODES, D)
    h = N_CODES // 2
    idx1 = _vq(codes[:h], cbt2, cn)
    idx2 = _vq(codes[h:], cbt2, cn)
    q1, parts1 = _sc_gather_hist(idx1.reshape(1, h), cb_pad)
    q2, parts2 = _sc_gather_hist(idx2.reshape(1, h), cb_pad)
    q = jnp.concatenate([q1[:, :D].reshape(B // 2, F),
                         q2[:, :D].reshape(B // 2, F)], axis=0)
    partials = jnp.concatenate([parts1, parts2], axis=0)
    recons, vq, ppl = _decode(q, latent, w3b, b3r, w4b, b4r, partials)
    return recons, vq.reshape(()), ppl.reshape(())


# speedup vs baseline: 1.0729x; 1.0729x over previous
import dataclasses
from functools import partial

import jax
import jax.numpy as jnp
from jax.experimental import pallas as pl
from jax.experimental.pallas import tpu as pltpu
from jax.experimental.pallas import tpu_sc as plsc

B = 4096
F = 2048
K = 1024
D = 64
NJ = F // D
N_CODES = B * NJ
BM = 512
GRID_M = B // BM
SC_CORES = 2
SC_SUBCORES = 16
SC_WORKERS = SC_CORES * SC_SUBCORES
GATHER_W = 256

_DOT_STD = (((1,), (0,)), ((), ()))
_DOT_TB = (((1,), (1,)), ((), ()))


def _encode_body(x_ref, w1_ref, b1_ref, w2_ref, b2_ref, lat_ref):
    xb = x_ref[...].astype(jnp.bfloat16)
    h = jax.lax.dot_general(xb, w1_ref[...], _DOT_TB,
                            preferred_element_type=jnp.float32) + b1_ref[...]
    hb = jnp.maximum(h, 0.0).astype(jnp.bfloat16)
    lat_ref[...] = jax.lax.dot_general(
        hb, w2_ref[...], _DOT_TB,
        preferred_element_type=jnp.float32) + b2_ref[...]


def _encode(x, w1b, b1, w2b, b2):
    return pl.pallas_call(
        _encode_body,
        grid=(GRID_M,),
        in_specs=[
            pl.BlockSpec((BM, F), lambda m: (m, 0)),
            pl.BlockSpec((F, F), lambda m: (0, 0)),
            pl.BlockSpec((1, F), lambda m: (0, 0)),
            pl.BlockSpec((F, F), lambda m: (0, 0)),
            pl.BlockSpec((1, F), lambda m: (0, 0)),
        ],
        out_specs=pl.BlockSpec((BM, F), lambda m: (m, 0)),
        out_shape=jax.ShapeDtypeStruct((B, F), jnp.float32),
        compiler_params=pltpu.CompilerParams(
            dimension_semantics=("arbitrary",)),
    )(x, w1b, b1, w2b, b2)


BC = 4096
VQ_STEPS = N_CODES // BC


VQ_RT = 256


def _vq_body(codes_ref, cbt2_ref, cn_ref, idx_ref):
    cbtb = cbt2_ref[...].astype(jnp.bfloat16)
    cn = cn_ref[...]
    for rt in range(BC // VQ_RT):
        cs = codes_ref[pl.ds(rt * VQ_RT, VQ_RT), :].astype(jnp.bfloat16)
        d = jax.lax.dot_general(cs, cbtb, (((1,), (0,)), ((), ())),
                                preferred_element_type=jnp.float32)
        ij = jnp.argmin(d + cn, axis=1).astype(jnp.int32)
        idx_ref[0, 0, pl.ds(rt * VQ_RT, VQ_RT)] = ij


def _vq(codes, cbt2, cn):
    steps = codes.shape[0] // BC
    return pl.pallas_call(
        _vq_body,
        grid=(steps,),
        in_specs=[
            pl.BlockSpec((BC, D), lambda i: (i, 0)),
            pl.BlockSpec((D, K), lambda i: (0, 0)),
            pl.BlockSpec((1, K), lambda i: (0, 0)),
        ],
        out_specs=pl.BlockSpec((1, 1, BC), lambda i: (i, 0, 0)),
        out_shape=jax.ShapeDtypeStruct((steps, 1, BC), jnp.int32),
        compiler_params=pltpu.CompilerParams(
            dimension_semantics=("arbitrary",)),
    )(codes, cbt2, cn)


def _sc_gather_hist(idx2d, cb_pad):
    cp = pltpu.CompilerParams()
    if "needs_layout_passes" in pltpu.CompilerParams.__dataclass_fields__:
        cp = dataclasses.replace(cp, needs_layout_passes=False)
    mesh = plsc.VectorSubcoreMesh(core_axis_name="c", subcore_axis_name="s")
    n_idx = idx2d.shape[1]
    out_types = (
        jax.ShapeDtypeStruct((n_idx, 128), jnp.float32),
        jax.ShapeDtypeStruct((SC_WORKERS, K), jnp.float32),
    )

    @partial(pl.kernel, out_type=out_types, mesh=mesh,
             scratch_types=[pltpu.VMEM((K,), jnp.float32)],
             compiler_params=cp)
    def k(cb_hbm, i_hbm, q_hbm, h_hbm, hist_ref):
        @pl.loop(0, K, step=16)
        def _(c):
            hist_ref[pl.ds(c, 16)] = jnp.zeros((16,), jnp.float32)

        def body(i_vmem, o_vmem):
            pltpu.sync_copy(cb_hbm.at[i_vmem.at[0]], o_vmem)

            @pl.loop(0, GATHER_W, step=16)
            def _(c):
                v = i_vmem[0, pl.ds(c, 16)]
                plsc.addupdate_scatter(hist_ref, [v],
                                       jnp.ones((16,), jnp.float32))

        pltpu.emit_pipeline(
            body,
            grid=(n_idx // GATHER_W,),
            in_specs=[pl.BlockSpec((1, GATHER_W), index_map=lambda i: (0, i))],
            out_specs=[pl.BlockSpec((GATHER_W, 128),
                                    index_map=lambda i: (i, 0))],
            core_axis_name=("c", "s"),
            dimension_semantics=(pltpu.PARALLEL,),
        )(i_hbm, q_hbm)

        sid = jax.lax.axis_index("c") * SC_SUBCORES + jax.lax.axis_index("s")
        pltpu.sync_copy(hist_ref, h_hbm.at[sid])

    return k(cb_pad, idx2d)


def _decode_body(q_ref, lat_ref, w3_ref, b3_ref, w4_ref, b4_ref, part_ref,
                 rec_ref, vq_ref, ppl_ref, acc_ref):
    m = pl.program_id(0)

    @pl.when(m == 0)
    def _():
        acc_ref[0] = 0.0

    qb = q_ref[...]
    diff = qb - lat_ref[...]
    acc_ref[0] += jnp.sum(diff * diff)

    h2 = jax.lax.dot_general(qb.astype(jnp.bfloat16), w3_ref[...], _DOT_TB,
                             preferred_element_type=jnp.float32) + b3_ref[...]
    h2b = jnp.maximum(h2, 0.0).astype(jnp.bfloat16)
    rec_ref[...] = jax.lax.dot_general(h2b, w4_ref[...], _DOT_TB,
                                       preferred_element_type=jnp.float32
                                       ) + b4_ref[...]

    @pl.when(m == GRID_M - 1)
    def _():
        vq_ref[...] = jnp.reshape(acc_ref[0] * (1.25 / float(N_CODES * D)),
                                  (1, 1))
        counts = jnp.sum(part_ref[...], axis=0)
        p = counts * (1.0 / float(N_CODES))
        ppl_ref[...] = jnp.reshape(jnp.exp(-jnp.sum(p * jnp.log(p + 1e-10))),
                                   (1, 1))


def _decode(q, latent, w3b, b3, w4b, b4, partials):
    return pl.pallas_call(
        _decode_body,
        grid=(GRID_M,),
        in_specs=[
            pl.BlockSpec((BM, F), lambda m: (m, 0)),
            pl.BlockSpec((BM, F), lambda m: (m, 0)),
            pl.BlockSpec((F, F), lambda m: (0, 0)),
            pl.BlockSpec((1, F), lambda m: (0, 0)),
            pl.BlockSpec((F, F), lambda m: (0, 0)),
            pl.BlockSpec((1, F), lambda m: (0, 0)),
            pl.BlockSpec((SC_WORKERS, K), lambda m: (0, 0)),
        ],
        out_specs=[
            pl.BlockSpec((BM, F), lambda m: (m, 0)),
            pl.BlockSpec((1, 1), lambda m: (0, 0)),
            pl.BlockSpec((1, 1), lambda m: (0, 0)),
        ],
        out_shape=[
            jax.ShapeDtypeStruct((B, F), jnp.float32),
            jax.ShapeDtypeStruct((1, 1), jnp.float32),
            jax.ShapeDtypeStruct((1, 1), jnp.float32),
        ],
        scratch_shapes=[pltpu.SMEM((1,), jnp.float32)],
        compiler_params=pltpu.CompilerParams(
            dimension_semantics=("arbitrary",)),
    )(q, latent, w3b, b3, w4b, b4, partials)


def kernel(x, W1, b1, W2, b2, W3, b3, W4, b4, codebook):
    w1b = W1.astype(jnp.bfloat16)
    w2b = W2.astype(jnp.bfloat16)
    w3b = W3.astype(jnp.bfloat16)
    w4b = W4.astype(jnp.bfloat16)
    b1r = b1.reshape(1, F)
    b2r = b2.reshape(1, F)
    b3r = b3.reshape(1, F)
    b4r = b4.reshape(1, F)

    cbt2 = codebook.T * jnp.float32(-2.0)
    cn = jnp.sum(codebook * codebook, axis=1)[None, :]
    cb_pad = jnp.pad(codebook, ((0, 0), (0, 128 - D)))

    latent = _encode(x, w1b, b1r, w2b, b2r)
    idx = _vq(latent.reshape(N_CODES, D), cbt2, cn)
    q_pad, partials = _sc_gather_hist(idx.reshape(1, N_CODES), cb_pad)
    q = q_pad[:, :D].reshape(B, F)
    recons, vq, ppl = _decode(q, latent, w3b, b3r, w4b, b4r, partials)
    return recons, vq.reshape(()), ppl.reshape(())
